# baseline (speedup 1.0000x reference)
import functools

import jax
import jax.numpy as jnp
from jax import lax
from jax.experimental import pallas as pl
from jax.experimental.pallas import tpu as pltpu
from jax.experimental.pallas import tpu_sc as plsc

_N_ROUNDS = 3


def _dot(a, b, ta=False, tb=False):
    dn = (((0 if ta else 1,), (1 if tb else 0,)), ((), ()))
    return jax.lax.dot_general(a, b, dn, precision=jax.lax.Precision.HIGHEST,
                               preferred_element_type=jnp.float32)


def _dot_d(a, b, ta=False, tb=False):
    return _dot(a, b, ta=ta, tb=tb)


def _pick_bd(n, mult=8):
    for bd in (256, 200, 128, 100, 64, 50, 40, 32, 16, 8):
        if n % bd == 0 and bd % mult == 0:
            return bd
    return n


def _matmul_body(x_ref, w_ref, o_ref):
    o_ref[...] = _dot_d(x_ref[...], w_ref[...])


def _matmul(x, w):
    n, kd = x.shape
    m = w.shape[1]
    bd = _pick_bd(n)
    return pl.pallas_call(
        _matmul_body,
        grid=(n // bd,),
        in_specs=[
            pl.BlockSpec((bd, kd), lambda i: (i, 0)),
            pl.BlockSpec((kd, m), lambda i: (0, 0)),
        ],
        out_specs=pl.BlockSpec((bd, m), lambda i: (i, 0)),
        out_shape=jax.ShapeDtypeStruct((n, m), jnp.float32),
    )(x, w)


def _expm1_neg(x):
    xs = jnp.minimum(x, 0.0)
    poly = xs * (1.0 + xs * (0.5 + xs * (1.0 / 6.0 + xs * (1.0 / 24.0
                 + xs * (1.0 / 120.0 + xs / 720.0)))))
    return jnp.where(xs > -0.5, poly, jnp.exp(xs) - 1.0)


def _finish_block(agg, dst, f1, f2):
    h = jnp.where(agg > 0, agg, _expm1_neg(agg)) + dst
    h = h + _dot_d(jnp.maximum(_dot_d(h, f1), 0.0), f2)
    mu = jnp.mean(h, axis=1, keepdims=True)
    var = jnp.mean((h - mu) ** 2, axis=1, keepdims=True)
    return (h - mu) / jnp.sqrt(var + 1e-6)


def _gat_row_body(dst_ref, k_ref, c_ref, wq_ref, f1_ref, f2_ref, o_ref):
    dst = dst_ref[...]
    k = k_ref[...]
    c = c_ref[...]
    q = _dot_d(dst, wq_ref[...])
    t = _dot(q, k, tb=True)
    t = jnp.where(t >= 0, t, 0.2 * t)
    edge = c > 0
    neg = jnp.float32(-1e30)
    m = jnp.max(jnp.where(edge, t, neg), axis=1, keepdims=True)
    p = c * jnp.exp(jnp.where(edge, t - m, neg))
    den = jnp.sum(p, axis=1, keepdims=True)
    agg = _dot(p / (den + 1e-9), k)
    o_ref[...] = _finish_block(agg, dst, f1_ref[...], f2_ref[...])


def _gat_col_body(dst_ref, k_ref, c_ref, wq_ref, f1_ref, f2_ref, o_ref):
    dst = dst_ref[...]
    k = k_ref[...]
    c = c_ref[...]
    q = _dot_d(dst, wq_ref[...])
    t = _dot(k, q, tb=True)
    t = jnp.where(t >= 0, t, 0.2 * t)
    edge = c > 0
    neg = jnp.float32(-1e30)
    m = jnp.max(jnp.where(edge, t, neg), axis=0, keepdims=True)
    p = c * jnp.exp(jnp.where(edge, t - m, neg))
    den = jnp.sum(p, axis=0, keepdims=True)
    agg = _dot(p / (den + 1e-9), k, ta=True)
    o_ref[...] = _finish_block(agg, dst, f1_ref[...], f2_ref[...])


def _gat_layer(dst_state, src_state, cmat, wq, wk, f1, f2, col_mask):
    nd, dd = dst_state.shape
    ns = src_state.shape[0]
    h = wk.shape[1]
    kmat = _matmul(src_state, wk)
    ffn = f1.shape[1]
    if col_mask:
        bd = _pick_bd(nd, mult=128)
        body = _gat_col_body
        c_spec = pl.BlockSpec((ns, bd), lambda i: (0, i))
    else:
        bd = _pick_bd(nd)
        body = _gat_row_body
        c_spec = pl.BlockSpec((bd, ns), lambda i: (i, 0))
    return pl.pallas_call(
        body,
        grid=(nd // bd,),
        in_specs=[
            pl.BlockSpec((bd, dd), lambda i: (i, 0)),
            pl.BlockSpec((ns, h), lambda i: (0, 0)),
            c_spec,
            pl.BlockSpec((dd, h), lambda i: (0, 0)),
            pl.BlockSpec((dd, ffn), lambda i: (0, 0)),
            pl.BlockSpec((ffn, dd), lambda i: (0, 0)),
        ],
        out_specs=pl.BlockSpec((bd, dd), lambda i: (i, 0)),
        out_shape=jax.ShapeDtypeStruct((nd, dd), jnp.float32),
    )(dst_state, kmat, cmat, wq, f1, f2)


def _build_counts_sc(edst, esrc, nd, ns_dim):
    e = edst.shape[0]
    info = plsc.get_sparse_core_info()
    nc, nsc = info.num_cores, info.num_subcores
    nwk = nc * nsc
    tot = nd * ns_dim
    sl = 102400
    rounds = -(-tot // (nwk * sl))
    pad_out = rounds * nwk * sl
    ch = 6400
    nch = e // ch
    assert nch * ch == e and ch % 8 == 0 and sl % 8 == 0
    groups = ch // 16

    @functools.partial(
        pl.kernel,
        out_type=jax.ShapeDtypeStruct((pad_out // 128, 128), jnp.float32),
        mesh=plsc.VectorSubcoreMesh(core_axis_name="c", subcore_axis_name="s"),
        scratch_types=[
            pltpu.VMEM((sl // 128 + 1, 128), jnp.float32),
            pltpu.VMEM((2, ch), jnp.int32),
            pltpu.VMEM((2, ch), jnp.int32),
            pltpu.SemaphoreType.DMA,
            pltpu.SemaphoreType.DMA,
        ],
        compiler_params=pltpu.CompilerParams(needs_layout_passes=False),
    )
    def build(edst_hbm, esrc_hbm, out_hbm, slice_v, ea, eb, sem0, sem1):
        c = lax.axis_index("c")
        s = lax.axis_index("s")
        w = s * nc + c
        sems = (sem0, sem1)
        ones16 = jnp.full((16,), 1.0, jnp.float32)

        def start(chk):
            b = chk % 2
            return (pltpu.async_copy(esrc_hbm.at[pl.ds(chk * ch, ch)],
                                     ea.at[b], sems[b]),
                    pltpu.async_copy(edst_hbm.at[pl.ds(chk * ch, ch)],
                                     eb.at[b], sems[b]))

        for r in range(rounds):
            lo = (r * nwk + w) * sl

            def zfill(i, carry):
                for j in range(8):
                    slice_v[i, pl.ds(j * 16, 16)] = jnp.zeros((16,), jnp.float32)
                return carry

            lax.fori_loop(0, sl // 128, zfill, 0)
            pend = start(0)
            for chk in range(nch):
                b = chk % 2
                nxt = start(chk + 1) if chk + 1 < nch else None
                pend[0].wait()
                pend[1].wait()

                def scan(i, carry):
                    g = pl.ds(i * 16, 16)
                    flat = eb[b, g] * ns_dim + ea[b, g]
                    local = flat - lo
                    inw = (local >= 0) & (local < sl)
                    local = jnp.where(inw, local, sl)
                    plsc.addupdate_scatter(
                        slice_v,
                        [lax.shift_right_logical(local, 7), local & 127],
                        ones16)
                    return carry

                lax.fori_loop(0, groups, scan, 0)
                pend = nxt
            pltpu.sync_copy(
                slice_v.at[pl.ds(0, sl // 128)],
                out_hbm.at[pl.ds(pl.multiple_of(lo // 128, 8), sl // 128)])

    flat = build(edst.astype(jnp.int32), esrc.astype(jnp.int32))
    return flat.reshape(-1)[:tot].reshape(nd, ns_dim)


def kernel(word_feat, sent_raw, edge_src, edge_dst, W_proj, Wq_ws, Wk_ws,
           F1_s, F2_s, Wq_sw, Wk_sw, F1_w, F2_w, W_head, b_head):
    nw = word_feat.shape[0]
    nw_pad = -(-nw // 1024) * 1024
    word_feat = jnp.pad(word_feat, ((0, nw_pad - nw), (0, 0)))
    c_sw = _build_counts_sc(edge_dst, edge_src, sent_raw.shape[0],
                            nw_pad)
    sent_feature = _matmul(sent_raw, W_proj)
    word_state = word_feat
    sent_state = _gat_layer(sent_feature, word_state, c_sw,
                            Wq_ws, Wk_ws, F1_s, F2_s, col_mask=False)
    for _ in range(_N_ROUNDS):
        word_state = _gat_layer(word_state, sent_state, c_sw,
                                Wq_sw, Wk_sw, F1_w, F2_w, col_mask=True)
        sent_state = _gat_layer(sent_state, word_state, c_sw,
                                Wq_ws, Wk_ws, F1_s, F2_s, col_mask=False)
    return _matmul(sent_state, W_head) + b_head
